# baseline (speedup 1.0000x reference)
import dataclasses
import functools

import jax
import jax.numpy as jnp
from jax import lax
from jax.experimental import pallas as pl
from jax.experimental.pallas import tpu as pltpu
from jax.experimental.pallas import tpu_sc as plsc

N = 10000
E = 160000
D = 256
C = 16
G = 64

NP = 10240
NW = 32
FS = D // NW
EP = 163840
BK = 128
CH = 2048
NCH = EP // CH
NACC = NP + 8

_CP = pltpu.CompilerParams()
if "needs_layout_passes" in pltpu.CompilerParams.__dataclass_fields__:
    _CP = dataclasses.replace(_CP, needs_layout_passes=False)
if "use_tc_tiling_on_sc" in pltpu.CompilerParams.__dataclass_fields__:
    _CP = dataclasses.replace(_CP, use_tc_tiling_on_sc=False)


def _sc_segment_sum(h8, srcp, dstp):
    mesh = plsc.VectorSubcoreMesh(core_axis_name="c", subcore_axis_name="s")

    @functools.partial(
        pl.kernel,
        mesh=mesh,
        compiler_params=_CP,
        out_type=jax.ShapeDtypeStruct((NW, NACC * FS), jnp.float32),
        scratch_types=[
            pltpu.VMEM((NACC * FS,), jnp.float32),
            pltpu.VMEM((CH,), jnp.int32),
            pltpu.VMEM((CH,), jnp.int32),
            pltpu.VMEM((CH, FS), jnp.float32),
            pltpu.SemaphoreType.DMA,
        ],
    )
    def seg_sum(h_hbm, src_hbm, dst_hbm, out_hbm,
                agg, src_v, dst_v, stage, sem):
        c = lax.axis_index("c")
        s = lax.axis_index("s")
        wid = c * 16 + s
        rowbase = wid * NP

        zero16 = jnp.zeros((16,), jnp.float32)

        @pl.loop(0, NACC * FS, step=16)
        def _(i):
            agg[pl.ds(i, 16)] = zero16

        lane = lax.iota(jnp.int32, 16)

        @pl.loop(0, NCH)
        def _(ch):
            pltpu.sync_copy(src_hbm.at[pl.ds(ch * CH, CH)], src_v)
            pltpu.sync_copy(dst_hbm.at[pl.ds(ch * CH, CH)], dst_v)
            @pl.loop(0, CH, step=16)
            def _(i):
                src_v[pl.ds(i, 16)] = src_v[pl.ds(i, 16)] + rowbase

            @pl.loop(0, CH // BK)
            def _(k):
                pltpu.async_copy(
                    h_hbm.at[src_v.at[pl.ds(k * BK, BK)]],
                    stage.at[pl.ds(k * BK, BK)], sem,
                ).wait()

            @pl.loop(0, CH, step=16)
            def _(g):
                eidx = lane + g
                didx8 = dst_v[pl.ds(g, 16)] * FS
                for f in range(FS):
                    fidx = jnp.full((16,), f, jnp.int32)
                    vals = plsc.load_gather(stage, [eidx, fidx])
                    plsc.addupdate_scatter(agg, [didx8 + f], vals)

        pltpu.sync_copy(agg, out_hbm.at[wid])

    return seg_sum(h8, srcp, dstp)


_PREC = lax.Precision.HIGHEST


def _mlp_block(h_ref, a_ref, w1_ref, b1_ref, w2_ref, b2_ref, o_ref, *, relu_out):
    z = h_ref[...] + a_ref[...]
    z = jnp.dot(z, w1_ref[...], preferred_element_type=jnp.float32,
                precision=_PREC) + b1_ref[...]
    z = jnp.maximum(z, 0.0)
    z = jnp.dot(z, w2_ref[...], preferred_element_type=jnp.float32,
                precision=_PREC) + b2_ref[...]
    if relu_out:
        z = jnp.maximum(z, 0.0)
    o_ref[...] = z


def _tc_mlp(h, agg, w1, b1, w2, b2, relu_out):
    bm = 256
    grid = (NP // bm,)
    return pl.pallas_call(
        functools.partial(_mlp_block, relu_out=relu_out),
        grid=grid,
        in_specs=[
            pl.BlockSpec((bm, D), lambda i: (i, 0)),
            pl.BlockSpec((bm, D), lambda i: (i, 0)),
            pl.BlockSpec((D, D), lambda i: (0, 0)),
            pl.BlockSpec((1, D), lambda i: (0, 0)),
            pl.BlockSpec((D, D), lambda i: (0, 0)),
            pl.BlockSpec((1, D), lambda i: (0, 0)),
        ],
        out_specs=pl.BlockSpec((bm, D), lambda i: (i, 0)),
        out_shape=jax.ShapeDtypeStruct((NP, D), jnp.float32),
    )(h, agg, w1, b1, w2, b2)


def _final_block(h_ref, a_ref, w1_ref, b1_ref, w2_ref, b2_ref, bid_ref,
                 o_ref, acc_ref):
    i = pl.program_id(0)
    nsteps = pl.num_programs(0)
    z = h_ref[...] + a_ref[...]
    z = jnp.dot(z, w1_ref[...], preferred_element_type=jnp.float32,
                precision=_PREC) + b1_ref[...]
    z = jnp.maximum(z, 0.0)
    z = jnp.dot(z, w2_ref[...], preferred_element_type=jnp.float32,
                precision=_PREC) + b2_ref[...]
    gids = lax.broadcasted_iota(jnp.int32, (z.shape[0], G), 1).astype(jnp.float32)
    onehot = (bid_ref[...] == gids).astype(jnp.float32)
    pb = lax.dot_general(onehot, z, (((0,), (0,)), ((), ())),
                         preferred_element_type=jnp.float32, precision=_PREC)

    @pl.when(i == 0)
    def _():
        acc_ref[...] = jnp.zeros_like(acc_ref)

    acc_ref[...] += pb

    @pl.when(i == nsteps - 1)
    def _():
        p = acc_ref[...]
        m = jnp.max(p, axis=-1, keepdims=True)
        e = jnp.exp(p - m)
        o_ref[...] = p - m - jnp.log(jnp.sum(e, axis=-1, keepdims=True))


def _tc_final(h, agg, w1, b1, w2, b2, bids):
    bm = 256
    grid = (NP // bm,)
    return pl.pallas_call(
        _final_block,
        grid=grid,
        in_specs=[
            pl.BlockSpec((bm, D), lambda i: (i, 0)),
            pl.BlockSpec((bm, D), lambda i: (i, 0)),
            pl.BlockSpec((D, C), lambda i: (0, 0)),
            pl.BlockSpec((1, C), lambda i: (0, 0)),
            pl.BlockSpec((C, C), lambda i: (0, 0)),
            pl.BlockSpec((1, C), lambda i: (0, 0)),
            pl.BlockSpec((bm, 1), lambda i: (i, 0)),
        ],
        out_specs=pl.BlockSpec((G, C), lambda i: (0, 0)),
        out_shape=jax.ShapeDtypeStruct((G, C), jnp.float32),
        scratch_shapes=[pltpu.VMEM((G, C), jnp.float32)],
    )(h, agg, w1, b1, w2, b2, bids)


def kernel(x, edge_index, batch,
           l0_w1, l0_b1, l0_w2, l0_b2,
           l1_w1, l1_b1, l1_w2, l1_b2,
           l2_w1, l2_b1, l2_w2, l2_b2,
           l3_w1, l3_b1, l3_w2, l3_b2):
    src = edge_index[0]
    dst = edge_index[1]
    srcp = jnp.concatenate([src, jnp.zeros((EP - E,), jnp.int32)])
    dstp = jnp.concatenate([dst, jnp.full((EP - E,), NP, jnp.int32)])
    hp = jnp.concatenate([x, jnp.zeros((NP - N, D), jnp.float32)])

    def to_h8(h):
        return h.reshape(NP, NW, FS).transpose(1, 0, 2).reshape(NW * NP, FS)

    def from_out(o):
        return o.reshape(NW, NACC, FS).transpose(1, 0, 2).reshape(NACC, D)
    bids = jnp.concatenate(
        [batch.astype(jnp.float32), jnp.full((NP - N,), float(G), jnp.float32)]
    ).reshape(NP, 1)

    params = [(l0_w1, l0_b1, l0_w2, l0_b2),
              (l1_w1, l1_b1, l1_w2, l1_b2),
              (l2_w1, l2_b1, l2_w2, l2_b2)]
    h = hp
    for w1, b1, w2, b2 in params:
        agg = from_out(_sc_segment_sum(to_h8(h), srcp, dstp))
        h = _tc_mlp(h, agg, w1, b1.reshape(1, D), w2, b2.reshape(1, D),
                    relu_out=True)
    agg = from_out(_sc_segment_sum(to_h8(h), srcp, dstp))
    return _tc_final(h, agg, l3_w1, l3_b1.reshape(1, C), l3_w2,
                     l3_b2.reshape(1, C), bids)


# speedup vs baseline: 2.7342x; 2.7342x over previous
import dataclasses
import functools

import jax
import jax.numpy as jnp
from jax import lax
from jax.experimental import pallas as pl
from jax.experimental.pallas import tpu as pltpu
from jax.experimental.pallas import tpu_sc as plsc

N = 10000
E = 160000
D = 256
C = 16
G = 64

NP = 10240
NW = 32
FS = D // NW
EP = 163840
CH = 2048
NCH = EP // CH
NACC = NP + 8

_CP = pltpu.CompilerParams()
if "needs_layout_passes" in pltpu.CompilerParams.__dataclass_fields__:
    _CP = dataclasses.replace(_CP, needs_layout_passes=False)
if "use_tc_tiling_on_sc" in pltpu.CompilerParams.__dataclass_fields__:
    _CP = dataclasses.replace(_CP, use_tc_tiling_on_sc=False)


def _sc_segment_sum(h8, srcp, dstp):
    mesh = plsc.VectorSubcoreMesh(core_axis_name="c", subcore_axis_name="s")

    @functools.partial(
        pl.kernel,
        mesh=mesh,
        compiler_params=_CP,
        out_type=jax.ShapeDtypeStruct((NW, NACC * FS), jnp.float32),
        scratch_types=[
            pltpu.VMEM((NACC * FS,), jnp.float32),
            pltpu.VMEM((CH,), jnp.int32),
            pltpu.VMEM((CH,), jnp.int32),
            pltpu.VMEM((CH,), jnp.int32),
            pltpu.VMEM((CH,), jnp.int32),
            pltpu.VMEM((CH, FS), jnp.float32),
            pltpu.VMEM((CH, FS), jnp.float32),
            pltpu.SemaphoreType.DMA,
            pltpu.SemaphoreType.DMA,
        ],
    )
    def seg_sum(h_hbm, src_hbm, dst_hbm, out_hbm,
                agg, src_a, dst_a, src_b, dst_b, stage_a, stage_b,
                sem_a, sem_b):
        c = lax.axis_index("c")
        s = lax.axis_index("s")
        wid = c * 16 + s
        rowbase = wid * NP

        zero16 = jnp.zeros((16,), jnp.float32)

        @plsc.parallel_loop(0, NACC * FS, step=16, unroll=4)
        def _(i):
            agg[pl.ds(i, 16)] = zero16

        lane = lax.iota(jnp.int32, 16)

        def fire(ch, src_v, dst_v, stage, sem):
            pltpu.sync_copy(src_hbm.at[pl.ds(ch * CH, CH)], src_v)
            pltpu.sync_copy(dst_hbm.at[pl.ds(ch * CH, CH)], dst_v)

            @plsc.parallel_loop(0, CH, step=16, unroll=4)
            def _(i):
                src_v[pl.ds(i, 16)] = src_v[pl.ds(i, 16)] + rowbase

            pltpu.async_copy(h_hbm.at[src_v], stage, sem)

        def drain_adds(src_v, dst_v, stage, sem):
            pltpu.make_async_copy(h_hbm.at[src_v], stage, sem).wait()

            @plsc.parallel_loop(0, CH, step=16, unroll=2)
            def _(g):
                eidx = lane + g
                didx8 = dst_v[pl.ds(g, 16)] * FS
                for f in range(FS):
                    fidx = jnp.full((16,), f, jnp.int32)
                    vals = plsc.load_gather(stage, [eidx, fidx])
                    plsc.addupdate_scatter(agg, [didx8 + f], vals)

        fire(0, src_a, dst_a, stage_a, sem_a)

        @pl.loop(0, NCH // 2 - 1)
        def _(p):
            ch = p * 2
            fire(ch + 1, src_b, dst_b, stage_b, sem_b)
            drain_adds(src_a, dst_a, stage_a, sem_a)
            fire(ch + 2, src_a, dst_a, stage_a, sem_a)
            drain_adds(src_b, dst_b, stage_b, sem_b)

        fire(NCH - 1, src_b, dst_b, stage_b, sem_b)
        drain_adds(src_a, dst_a, stage_a, sem_a)
        drain_adds(src_b, dst_b, stage_b, sem_b)

        pltpu.sync_copy(agg, out_hbm.at[wid])

    return seg_sum(h8, srcp, dstp)


_PREC = lax.Precision.HIGHEST


def _mlp_block(h_ref, a_ref, w1_ref, b1_ref, w2_ref, b2_ref, o_ref, *, relu_out):
    z = h_ref[...] + a_ref[...]
    z = jnp.dot(z, w1_ref[...], preferred_element_type=jnp.float32,
                precision=_PREC) + b1_ref[...]
    z = jnp.maximum(z, 0.0)
    z = jnp.dot(z, w2_ref[...], preferred_element_type=jnp.float32,
                precision=_PREC) + b2_ref[...]
    if relu_out:
        z = jnp.maximum(z, 0.0)
    o_ref[...] = z


def _tc_mlp(h, agg, w1, b1, w2, b2, relu_out):
    bm = 256
    grid = (NP // bm,)
    return pl.pallas_call(
        functools.partial(_mlp_block, relu_out=relu_out),
        grid=grid,
        in_specs=[
            pl.BlockSpec((bm, D), lambda i: (i, 0)),
            pl.BlockSpec((bm, D), lambda i: (i, 0)),
            pl.BlockSpec((D, D), lambda i: (0, 0)),
            pl.BlockSpec((1, D), lambda i: (0, 0)),
            pl.BlockSpec((D, D), lambda i: (0, 0)),
            pl.BlockSpec((1, D), lambda i: (0, 0)),
        ],
        out_specs=pl.BlockSpec((bm, D), lambda i: (i, 0)),
        out_shape=jax.ShapeDtypeStruct((NP, D), jnp.float32),
    )(h, agg, w1, b1, w2, b2)


def _final_block(h_ref, a_ref, w1_ref, b1_ref, w2_ref, b2_ref, bid_ref,
                 o_ref, acc_ref):
    i = pl.program_id(0)
    nsteps = pl.num_programs(0)
    z = h_ref[...] + a_ref[...]
    z = jnp.dot(z, w1_ref[...], preferred_element_type=jnp.float32,
                precision=_PREC) + b1_ref[...]
    z = jnp.maximum(z, 0.0)
    z = jnp.dot(z, w2_ref[...], preferred_element_type=jnp.float32,
                precision=_PREC) + b2_ref[...]
    gids = lax.broadcasted_iota(jnp.int32, (z.shape[0], G), 1).astype(jnp.float32)
    onehot = (bid_ref[...] == gids).astype(jnp.float32)
    pb = lax.dot_general(onehot, z, (((0,), (0,)), ((), ())),
                         preferred_element_type=jnp.float32, precision=_PREC)

    @pl.when(i == 0)
    def _():
        acc_ref[...] = jnp.zeros_like(acc_ref)

    acc_ref[...] += pb

    @pl.when(i == nsteps - 1)
    def _():
        p = acc_ref[...]
        m = jnp.max(p, axis=-1, keepdims=True)
        e = jnp.exp(p - m)
        o_ref[...] = p - m - jnp.log(jnp.sum(e, axis=-1, keepdims=True))


def _tc_final(h, agg, w1, b1, w2, b2, bids):
    bm = 256
    grid = (NP // bm,)
    return pl.pallas_call(
        _final_block,
        grid=grid,
        in_specs=[
            pl.BlockSpec((bm, D), lambda i: (i, 0)),
            pl.BlockSpec((bm, D), lambda i: (i, 0)),
            pl.BlockSpec((D, C), lambda i: (0, 0)),
            pl.BlockSpec((1, C), lambda i: (0, 0)),
            pl.BlockSpec((C, C), lambda i: (0, 0)),
            pl.BlockSpec((1, C), lambda i: (0, 0)),
            pl.BlockSpec((bm, 1), lambda i: (i, 0)),
        ],
        out_specs=pl.BlockSpec((G, C), lambda i: (0, 0)),
        out_shape=jax.ShapeDtypeStruct((G, C), jnp.float32),
        scratch_shapes=[pltpu.VMEM((G, C), jnp.float32)],
    )(h, agg, w1, b1, w2, b2, bids)


def kernel(x, edge_index, batch,
           l0_w1, l0_b1, l0_w2, l0_b2,
           l1_w1, l1_b1, l1_w2, l1_b2,
           l2_w1, l2_b1, l2_w2, l2_b2,
           l3_w1, l3_b1, l3_w2, l3_b2):
    src = edge_index[0]
    dst = edge_index[1]
    srcp = jnp.concatenate([src, jnp.zeros((EP - E,), jnp.int32)])
    dstp = jnp.concatenate([dst, jnp.full((EP - E,), NP, jnp.int32)])
    hp = jnp.concatenate([x, jnp.zeros((NP - N, D), jnp.float32)])

    def to_h8(h):
        return h.reshape(NP, NW, FS).transpose(1, 0, 2).reshape(NW * NP, FS)

    def from_out(o):
        return o.reshape(NW, NACC, FS).transpose(1, 0, 2).reshape(NACC, D)
    bids = jnp.concatenate(
        [batch.astype(jnp.float32), jnp.full((NP - N,), float(G), jnp.float32)]
    ).reshape(NP, 1)

    params = [(l0_w1, l0_b1, l0_w2, l0_b2),
              (l1_w1, l1_b1, l1_w2, l1_b2),
              (l2_w1, l2_b1, l2_w2, l2_b2)]
    h = hp
    for w1, b1, w2, b2 in params:
        agg = from_out(_sc_segment_sum(to_h8(h), srcp, dstp))
        h = _tc_mlp(h, agg, w1, b1.reshape(1, D), w2, b2.reshape(1, D),
                    relu_out=True)
    agg = from_out(_sc_segment_sum(to_h8(h), srcp, dstp))
    return _tc_final(h, agg, l3_w1, l3_b1.reshape(1, C), l3_w2,
                     l3_b2.reshape(1, C), bids)


# speedup vs baseline: 3.1989x; 1.1700x over previous
import dataclasses
import functools

import jax
import jax.numpy as jnp
from jax import lax
from jax.experimental import pallas as pl
from jax.experimental.pallas import tpu as pltpu
from jax.experimental.pallas import tpu_sc as plsc

N = 10000
E = 160000
D = 256
C = 16
G = 64

NP = 10240
NW = 32
FS = D // NW
EP = 163840
CH = 2048
NCH = EP // CH
NACC = NP + 8

_CP = pltpu.CompilerParams()
if "needs_layout_passes" in pltpu.CompilerParams.__dataclass_fields__:
    _CP = dataclasses.replace(_CP, needs_layout_passes=False)
if "use_tc_tiling_on_sc" in pltpu.CompilerParams.__dataclass_fields__:
    _CP = dataclasses.replace(_CP, use_tc_tiling_on_sc=False)


def _sc_segment_sum(h8, srcp, dstp):
    mesh = plsc.VectorSubcoreMesh(core_axis_name="c", subcore_axis_name="s")

    @functools.partial(
        pl.kernel,
        mesh=mesh,
        compiler_params=_CP,
        out_type=jax.ShapeDtypeStruct((NW, NACC * FS), jnp.float32),
        scratch_types=[
            pltpu.VMEM((NACC * FS,), jnp.float32),
            pltpu.VMEM((CH,), jnp.int32),
            pltpu.VMEM((CH,), jnp.int32),
            pltpu.VMEM((CH,), jnp.int32),
            pltpu.VMEM((CH,), jnp.int32),
            pltpu.VMEM((CH, FS), jnp.float32),
            pltpu.VMEM((CH, FS), jnp.float32),
            pltpu.SemaphoreType.DMA,
            pltpu.SemaphoreType.DMA,
            pltpu.SemaphoreType.DMA,
            pltpu.SemaphoreType.DMA,
            pltpu.SemaphoreType.DMA,
            pltpu.SemaphoreType.DMA,
        ],
    )
    def seg_sum(h_hbm, src_hbm, dst_hbm, out_hbm,
                agg, src_a, dst_a, src_b, dst_b, stage_a, stage_b,
                sem_a, sem_b, ssem_a, ssem_b, dsem_a, dsem_b):
        c = lax.axis_index("c")
        s = lax.axis_index("s")
        wid = c * 16 + s
        ebase = wid * EP

        zero16 = jnp.zeros((16,), jnp.float32)

        @plsc.parallel_loop(0, NACC * FS, step=16, unroll=4)
        def _(i):
            agg[pl.ds(i, 16)] = zero16

        lane = lax.iota(jnp.int32, 16)

        def start_src(ch, src_v, ssem):
            pltpu.async_copy(src_hbm.at[pl.ds(ebase + ch * CH, CH)], src_v,
                             ssem)

        def start_dst(ch, dst_v, dsem):
            pltpu.async_copy(dst_hbm.at[pl.ds(ch * CH, CH)], dst_v, dsem)

        def fire_gather(ch, src_v, stage, sem, ssem):
            pltpu.make_async_copy(
                src_hbm.at[pl.ds(ebase + ch * CH, CH)], src_v, ssem).wait()
            pltpu.async_copy(h_hbm.at[src_v], stage, sem)

        def wait_gather(src_v, stage, sem):
            pltpu.make_async_copy(h_hbm.at[src_v], stage, sem).wait()

        def adds(ch, dst_v, stage, dsem):
            pltpu.make_async_copy(
                dst_hbm.at[pl.ds(ch * CH, CH)], dst_v, dsem).wait()

            @plsc.parallel_loop(0, CH, step=16, unroll=2)
            def _(g):
                eidx = lane + g
                didx8 = dst_v[pl.ds(g, 16)] * FS
                for f in range(FS):
                    fidx = jnp.full((16,), f, jnp.int32)
                    vals = plsc.load_gather(stage, [eidx, fidx])
                    plsc.addupdate_scatter(agg, [didx8 + f], vals)

        start_src(0, src_a, ssem_a)
        start_dst(0, dst_a, dsem_a)
        fire_gather(0, src_a, stage_a, sem_a, ssem_a)
        start_src(1, src_b, ssem_b)
        start_dst(1, dst_b, dsem_b)

        @pl.loop(0, NCH // 2 - 1)
        def _(p):
            ch = p * 2
            fire_gather(ch + 1, src_b, stage_b, sem_b, ssem_b)
            wait_gather(src_a, stage_a, sem_a)
            start_src(ch + 2, src_a, ssem_a)
            adds(ch, dst_a, stage_a, dsem_a)
            start_dst(ch + 2, dst_a, dsem_a)
            fire_gather(ch + 2, src_a, stage_a, sem_a, ssem_a)
            wait_gather(src_b, stage_b, sem_b)
            start_src(ch + 3, src_b, ssem_b)
            adds(ch + 1, dst_b, stage_b, dsem_b)
            start_dst(ch + 3, dst_b, dsem_b)

        fire_gather(NCH - 1, src_b, stage_b, sem_b, ssem_b)
        wait_gather(src_a, stage_a, sem_a)
        adds(NCH - 2, dst_a, stage_a, dsem_a)
        wait_gather(src_b, stage_b, sem_b)
        adds(NCH - 1, dst_b, stage_b, dsem_b)

        pltpu.sync_copy(agg, out_hbm.at[wid])

    return seg_sum(h8, srcp, dstp)


_PREC = lax.Precision.HIGHEST


def _mlp_block(h_ref, a_ref, w1_ref, b1_ref, w2_ref, b2_ref, o_ref, *, relu_out):
    z = h_ref[...] + a_ref[...]
    z = jnp.dot(z, w1_ref[...], preferred_element_type=jnp.float32,
                precision=_PREC) + b1_ref[...]
    z = jnp.maximum(z, 0.0)
    z = jnp.dot(z, w2_ref[...], preferred_element_type=jnp.float32,
                precision=_PREC) + b2_ref[...]
    if relu_out:
        z = jnp.maximum(z, 0.0)
    o_ref[...] = z


def _tc_mlp(h, agg, w1, b1, w2, b2, relu_out):
    bm = 256
    grid = (NP // bm,)
    return pl.pallas_call(
        functools.partial(_mlp_block, relu_out=relu_out),
        grid=grid,
        in_specs=[
            pl.BlockSpec((bm, D), lambda i: (i, 0)),
            pl.BlockSpec((bm, D), lambda i: (i, 0)),
            pl.BlockSpec((D, D), lambda i: (0, 0)),
            pl.BlockSpec((1, D), lambda i: (0, 0)),
            pl.BlockSpec((D, D), lambda i: (0, 0)),
            pl.BlockSpec((1, D), lambda i: (0, 0)),
        ],
        out_specs=pl.BlockSpec((bm, D), lambda i: (i, 0)),
        out_shape=jax.ShapeDtypeStruct((NP, D), jnp.float32),
    )(h, agg, w1, b1, w2, b2)


def _final_block(h_ref, a_ref, w1_ref, b1_ref, w2_ref, b2_ref, bid_ref,
                 o_ref, acc_ref):
    i = pl.program_id(0)
    nsteps = pl.num_programs(0)
    z = h_ref[...] + a_ref[...]
    z = jnp.dot(z, w1_ref[...], preferred_element_type=jnp.float32,
                precision=_PREC) + b1_ref[...]
    z = jnp.maximum(z, 0.0)
    z = jnp.dot(z, w2_ref[...], preferred_element_type=jnp.float32,
                precision=_PREC) + b2_ref[...]
    gids = lax.broadcasted_iota(jnp.int32, (z.shape[0], G), 1).astype(jnp.float32)
    onehot = (bid_ref[...] == gids).astype(jnp.float32)
    pb = lax.dot_general(onehot, z, (((0,), (0,)), ((), ())),
                         preferred_element_type=jnp.float32, precision=_PREC)

    @pl.when(i == 0)
    def _():
        acc_ref[...] = jnp.zeros_like(acc_ref)

    acc_ref[...] += pb

    @pl.when(i == nsteps - 1)
    def _():
        p = acc_ref[...]
        m = jnp.max(p, axis=-1, keepdims=True)
        e = jnp.exp(p - m)
        o_ref[...] = p - m - jnp.log(jnp.sum(e, axis=-1, keepdims=True))


def _tc_final(h, agg, w1, b1, w2, b2, bids):
    bm = 256
    grid = (NP // bm,)
    return pl.pallas_call(
        _final_block,
        grid=grid,
        in_specs=[
            pl.BlockSpec((bm, D), lambda i: (i, 0)),
            pl.BlockSpec((bm, D), lambda i: (i, 0)),
            pl.BlockSpec((D, C), lambda i: (0, 0)),
            pl.BlockSpec((1, C), lambda i: (0, 0)),
            pl.BlockSpec((C, C), lambda i: (0, 0)),
            pl.BlockSpec((1, C), lambda i: (0, 0)),
            pl.BlockSpec((bm, 1), lambda i: (i, 0)),
        ],
        out_specs=pl.BlockSpec((G, C), lambda i: (0, 0)),
        out_shape=jax.ShapeDtypeStruct((G, C), jnp.float32),
        scratch_shapes=[pltpu.VMEM((G, C), jnp.float32)],
    )(h, agg, w1, b1, w2, b2, bids)


def kernel(x, edge_index, batch,
           l0_w1, l0_b1, l0_w2, l0_b2,
           l1_w1, l1_b1, l1_w2, l1_b2,
           l2_w1, l2_b1, l2_w2, l2_b2,
           l3_w1, l3_b1, l3_w2, l3_b2):
    src = edge_index[0]
    dst = edge_index[1]
    srcp = jnp.concatenate([src, jnp.zeros((EP - E,), jnp.int32)])
    srcp32 = (srcp[None, :]
              + (jnp.arange(NW, dtype=jnp.int32) * NP)[:, None]).reshape(-1)
    dstp = jnp.concatenate([dst, jnp.full((EP - E,), NP, jnp.int32)])
    hp = jnp.concatenate([x, jnp.zeros((NP - N, D), jnp.float32)])

    def to_h8(h):
        return h.reshape(NP, NW, FS).transpose(1, 0, 2).reshape(NW * NP, FS)

    def from_out(o):
        return o.reshape(NW, NACC, FS).transpose(1, 0, 2).reshape(NACC, D)
    bids = jnp.concatenate(
        [batch.astype(jnp.float32), jnp.full((NP - N,), float(G), jnp.float32)]
    ).reshape(NP, 1)

    params = [(l0_w1, l0_b1, l0_w2, l0_b2),
              (l1_w1, l1_b1, l1_w2, l1_b2),
              (l2_w1, l2_b1, l2_w2, l2_b2)]
    h = hp
    for w1, b1, w2, b2 in params:
        agg = from_out(_sc_segment_sum(to_h8(h), srcp32, dstp))
        h = _tc_mlp(h, agg, w1, b1.reshape(1, D), w2, b2.reshape(1, D),
                    relu_out=True)
    agg = from_out(_sc_segment_sum(to_h8(h), srcp32, dstp))
    return _tc_final(h, agg, l3_w1, l3_b1.reshape(1, C), l3_w2,
                     l3_b2.reshape(1, C), bids)
